# baseline (device time: 84182 ns/iter reference)
import jax
import jax.numpy as jnp
from jax import lax
from jax.experimental import pallas as pl
from jax.experimental.pallas import tpu as pltpu

N_Y = 4
N_X = 2
N_Z = 4
N_REP = N_X * N_Z
N_H = 2


def kernel(x, dy):
    m_per, d = x.shape
    _, f = dy.shape
    chunk = d // N_Y
    fsl = f // N_REP
    fh = fsl // N_H

    def body(x_ref, dy_ref, out_ref, r_buf, l_buf,
             rz_recv_sems, lz_recv_sems, rz_send_sems, lz_send_sems,
             z_recv_sems, x_recv_sems, own_send_sems, relay_send_sems):
        my_x = lax.axis_index("x")
        my_y = lax.axis_index("y")
        my_z = lax.axis_index("z")
        rep = my_x * N_Z + my_z
        xp = 1 - my_x
        has_l = my_z > 0
        has_r = my_z < N_Z - 1
        has_yl = my_y > 0
        has_yr = my_y < N_Y - 1
        zl = jnp.maximum(my_z - 1, 0)
        zr = jnp.minimum(my_z + 1, N_Z - 1)
        yl = jnp.maximum(my_y - 1, 0)
        yr = jnp.minimum(my_y + 1, N_Y - 1)

        barrier_sem = pltpu.get_barrier_semaphore()
        pl.semaphore_signal(
            barrier_sem, inc=1,
            device_id=(xp, my_y, my_z),
            device_id_type=pl.DeviceIdType.MESH,
        )
        for cond, dev in (
            (has_yl, (my_x, yl, my_z)),
            (has_yr, (my_x, yr, my_z)),
            (has_l, (my_x, my_y, zl)),
            (has_r, (my_x, my_y, zr)),
        ):
            @pl.when(cond)
            def _(dev=dev):
                pl.semaphore_signal(
                    barrier_sem, inc=1,
                    device_id=dev,
                    device_id_type=pl.DeviceIdType.MESH,
                )
        n_nbr = (1 + has_yl.astype(jnp.int32) + has_yr.astype(jnp.int32)
                 + has_l.astype(jnp.int32) + has_r.astype(jnp.int32))
        pl.semaphore_wait(barrier_sem, n_nbr)

        def partial(c, h):
            return lax.dot_general(
                x_ref[:, pl.ds(c * chunk, chunk)],
                dy_ref[:, pl.ds(rep * fsl + h * fh, fh)],
                (((0,), (0,)), ((), ())),
                preferred_element_type=jnp.float32,
            )

        def piece_ref(j, h):
            return out_ref.at[:, pl.ds((my_x * N_Z + j) * fsl + h * fh, fh)]

        def xline_piece_ref(j, h):
            return out_ref.at[:, pl.ds((xp * N_Z + j) * fsl + h * fh, fh)]

        pending = []
        always_pending = []

        def send(src, dst, send_sem, recv_sem, dev, cond=None):
            rdma = pltpu.make_async_remote_copy(
                src_ref=src, dst_ref=dst, send_sem=send_sem,
                recv_sem=recv_sem, device_id=dev,
                device_id_type=pl.DeviceIdType.MESH,
            )
            if cond is None:
                rdma.start()
                always_pending.append(rdma)
            else:
                @pl.when(cond)
                def _():
                    rdma.start()
                pending.append((cond, rdma))

        def recv_wait(dst, recv_sem, dev, cond=None):
            rdma = pltpu.make_async_remote_copy(
                src_ref=dst, dst_ref=dst, send_sem=recv_sem,
                recv_sem=recv_sem, device_id=dev,
                device_id_type=pl.DeviceIdType.MESH,
            )
            if cond is None:
                rdma.wait_recv()
            else:
                @pl.when(cond)
                def _():
                    rdma.wait_recv()

        def phase1(h):
            p_all = [partial(q, h) for q in range(N_Y)]

            for q in range(N_Y - 1, 0, -1):
                cstart = my_y == 0

                @pl.when(cstart)
                def _(q=q, h=h):
                    r_buf[h, q] = p_all[q]
                send(r_buf.at[h, q], r_buf.at[h, q],
                     rz_send_sems.at[h, q], rz_recv_sems.at[h, q],
                     (my_x, yr, my_z), cond=cstart)
            for q in range(N_Y - 1):
                cstart = my_y == N_Y - 1

                @pl.when(cstart)
                def _(q=q, h=h):
                    l_buf[h, q] = p_all[q]
                send(l_buf.at[h, q], l_buf.at[h, q],
                     lz_send_sems.at[h, q], lz_recv_sems.at[h, q],
                     (my_x, yl, my_z), cond=cstart)

            for dd in range(N_Y):
                qr = N_Y - 1 - dd
                ql = dd
                rcond = has_yl & (qr >= my_y)
                recv_wait(r_buf.at[h, qr], rz_recv_sems.at[h, qr],
                          (my_x, yl, my_z), cond=rcond)
                rfwd = has_yl & (qr > my_y)

                @pl.when(rfwd)
                def _(qr=qr, h=h):
                    r_buf[h, qr] = r_buf[h, qr] + p_all[qr]
                send(r_buf.at[h, qr], r_buf.at[h, qr],
                     rz_send_sems.at[h, qr], rz_recv_sems.at[h, qr],
                     (my_x, yr, my_z), cond=rfwd)

                lcond = has_yr & (ql <= my_y)
                recv_wait(l_buf.at[h, ql], lz_recv_sems.at[h, ql],
                          (my_x, yr, my_z), cond=lcond)
                lfwd = has_yr & (ql < my_y)

                @pl.when(lfwd)
                def _(ql=ql, h=h):
                    l_buf[h, ql] = l_buf[h, ql] + p_all[ql]
                send(l_buf.at[h, ql], l_buf.at[h, ql],
                     lz_send_sems.at[h, ql], lz_recv_sems.at[h, ql],
                     (my_x, yl, my_z), cond=lfwd)

            acc = p_all[0]
            for q in range(1, N_Y):
                acc = jnp.where(my_y == q, p_all[q], acc)
            acc = acc + jnp.where(has_yl, r_buf[h, my_y], 0.0)
            acc = acc + jnp.where(has_yr, l_buf[h, my_y], 0.0)
            out_ref[:, pl.ds(rep * fsl + h * fh, fh)] = acc

        def phase2_send_own(h):
            send(piece_ref(my_z, h), piece_ref(my_z, h),
                 own_send_sems.at[h, 2], x_recv_sems.at[h, my_z],
                 (xp, my_y, my_z))
            send(piece_ref(my_z, h), piece_ref(my_z, h),
                 own_send_sems.at[h, 0], z_recv_sems.at[h, my_z],
                 (my_x, my_y, zl), cond=has_l)
            send(piece_ref(my_z, h), piece_ref(my_z, h),
                 own_send_sems.at[h, 1], z_recv_sems.at[h, my_z],
                 (my_x, my_y, zr), cond=has_r)

        def phase2_relay(h):
            for dd in range(1, N_Z):
                d = jnp.int32(dd)
                fl = my_z >= d
                jl = jnp.maximum(my_z - d, 0)
                recv_wait(piece_ref(jl, h), z_recv_sems.at[h, jl],
                          (my_x, my_y, zl), cond=fl)
                send(piece_ref(jl, h), piece_ref(jl, h),
                     relay_send_sems.at[h, jl, 0], z_recv_sems.at[h, jl],
                     (my_x, my_y, zr), cond=fl & has_r)
                send(piece_ref(jl, h), piece_ref(jl, h),
                     relay_send_sems.at[h, jl, 1], x_recv_sems.at[h, jl],
                     (xp, my_y, my_z), cond=fl)
                fr = my_z + d <= N_Z - 1
                jr = jnp.minimum(my_z + d, N_Z - 1)
                recv_wait(piece_ref(jr, h), z_recv_sems.at[h, jr],
                          (my_x, my_y, zr), cond=fr)
                send(piece_ref(jr, h), piece_ref(jr, h),
                     relay_send_sems.at[h, jr, 0], z_recv_sems.at[h, jr],
                     (my_x, my_y, zl), cond=fr & has_l)
                send(piece_ref(jr, h), piece_ref(jr, h),
                     relay_send_sems.at[h, jr, 1], x_recv_sems.at[h, jr],
                     (xp, my_y, my_z), cond=fr)

        def phase2_xwait(h):
            for j in range(N_Z):
                recv_wait(xline_piece_ref(j, h), x_recv_sems.at[h, j],
                          (xp, my_y, my_z))

        for h in range(N_H):
            phase1(h)
            phase2_send_own(h)
        for h in range(N_H):
            phase2_relay(h)
        for h in range(N_H):
            phase2_xwait(h)
        for cond, rdma in pending:
            @pl.when(cond)
            def _(rdma=rdma):
                rdma.wait_send()
        for rdma in always_pending:
            rdma.wait_send()

    return pl.pallas_call(
        body,
        out_shape=jax.ShapeDtypeStruct((chunk, f), jnp.float32),
        in_specs=[
            pl.BlockSpec(memory_space=pltpu.VMEM),
            pl.BlockSpec(memory_space=pltpu.VMEM),
        ],
        out_specs=pl.BlockSpec(memory_space=pltpu.VMEM),
        scratch_shapes=[
            pltpu.VMEM((N_H, N_Y, chunk, fh), jnp.float32),
            pltpu.VMEM((N_H, N_Y, chunk, fh), jnp.float32),
            pltpu.SemaphoreType.DMA((N_H, N_Y)),
            pltpu.SemaphoreType.DMA((N_H, N_Y)),
            pltpu.SemaphoreType.DMA((N_H, N_Y)),
            pltpu.SemaphoreType.DMA((N_H, N_Y)),
            pltpu.SemaphoreType.DMA((N_H, N_Z)),
            pltpu.SemaphoreType.DMA((N_H, N_Z)),
            pltpu.SemaphoreType.DMA((N_H, 3)),
            pltpu.SemaphoreType.DMA((N_H, N_Z, 2)),
        ],
        compiler_params=pltpu.CompilerParams(
            collective_id=0,
            vmem_limit_bytes=64 * 1024 * 1024,
        ),
    )(x, dy)
